# baseline (speedup 1.0000x reference)
import functools

import jax
import jax.numpy as jnp
from jax import lax
from jax.experimental import pallas as pl
from jax.experimental.pallas import tpu as pltpu
from jax.experimental.pallas import tpu_sc as plsc

B, D, M, K, BG, P = 256, 3072, 20000, 128, 2048, 128
NS = 16
NW = 32
HALF = M // 2
CP_ROWS = 624
CP_TAIL = HALF - (NS - 1) * CP_ROWS
GPW = K // NS
JPS = BG // NS


def _sac_row(w):
    return (w // NS) * HALF + (w % NS) * CP_ROWS


def _src_body(wi_col_ref, wi_row_ref, src_ref, winr_ref):
    col = wi_col_ref[...]
    row = wi_row_ref[...]
    eq = col == row
    jp = lax.broadcasted_iota(jnp.int32, (BG, 128), 0)
    src_ref[...] = jnp.max(jnp.where(eq, jp, -1), axis=0, keepdims=True)
    w = lax.broadcasted_iota(jnp.int32, (1, NW), 1)
    rw = (w // NS) * HALF + (w % NS) * CP_ROWS
    eq2 = col == rw
    jp2 = lax.broadcasted_iota(jnp.int32, (BG, NW), 0)
    winr_ref[...] = jnp.max(jnp.where(eq2, jp2, -1), axis=0, keepdims=True)


_src_call = pl.pallas_call(
    _src_body,
    grid=(BG // 128,),
    in_specs=[
        pl.BlockSpec((BG, 1), lambda i: (0, 0)),
        pl.BlockSpec((1, 128), lambda i: (0, i)),
    ],
    out_specs=[
        pl.BlockSpec((1, 128), lambda i: (0, i)),
        pl.BlockSpec((1, NW), lambda i: (0, 0)),
    ],
    out_shape=[
        jax.ShapeDtypeStruct((1, BG), jnp.int32),
        jax.ShapeDtypeStruct((1, NW), jnp.int32),
    ],
)


def _fix_body(winr_ref, mem_row_ref, img_row_ref, fix_ref):
    w = pl.program_id(0)
    wv = winr_ref[w]
    fix_ref[...] = jnp.where(wv >= 0, img_row_ref[...], mem_row_ref[...])


_fix_call = pl.pallas_call(
    _fix_body,
    grid_spec=pltpu.PrefetchScalarGridSpec(
        num_scalar_prefetch=1,
        grid=(NW,),
        in_specs=[
            pl.BlockSpec(
                (1, 1, D),
                lambda w, winr: ((w // NS) * HALF + (w % NS) * CP_ROWS, 0, 0)),
            pl.BlockSpec(
                (1, 1, D),
                lambda w, winr: (jnp.maximum(winr[w], 0), 0, 0)),
        ],
        out_specs=pl.BlockSpec((1, 1, D), lambda w, winr: (w, 0, 0)),
    ),
    out_shape=jax.ShapeDtypeStruct((NW, 1, D), jnp.float32),
)


def _loss_body(x1_ref, x2_ref, buf_ref, w_ref, out_ref):
    w = w_ref[...]
    z1 = jnp.dot(x1_ref[...], w, preferred_element_type=jnp.float32)
    z2 = jnp.dot(x2_ref[...], w, preferred_element_type=jnp.float32)
    zb = jnp.dot(buf_ref[...], w, preferred_element_type=jnp.float32)
    n1 = jnp.sqrt(jnp.sum(z1 * z1, axis=-1, keepdims=True)) + 1e-8
    n2 = jnp.sqrt(jnp.sum(z2 * z2, axis=-1, keepdims=True)) + 1e-8
    nb = jnp.sqrt(jnp.sum(zb * zb, axis=-1, keepdims=True)) + 1e-8
    ax = jnp.sum(z1 * z2, axis=-1, keepdims=True) / (n1 * n2)
    ab = jnp.sum(zb * zb, axis=-1, keepdims=True) / (nb * nb)
    tot = jnp.sum(ax) + 0.5 * jnp.sum(ab)
    out_ref[...] = jnp.broadcast_to(-tot / (B + 0.5 * K), (1, 1))


_loss_call = pl.pallas_call(
    _loss_body,
    in_specs=[
        pl.BlockSpec((B, D), lambda: (0, 0)),
        pl.BlockSpec((B, D), lambda: (0, 0)),
        pl.BlockSpec((K, D), lambda: (0, 0)),
        pl.BlockSpec((D, P), lambda: (0, 0)),
    ],
    out_specs=pl.BlockSpec((1, 1), lambda: (0, 0)),
    out_shape=jax.ShapeDtypeStruct((1, 1), jnp.float32),
)


def _sc_body(mem_hbm, img_hbm, gidx_hbm, src_hbm, tgt_hbm, fix_hbm,
             buf_out, nm_hbm,
             gidx_v, grow_v, tgt_v, src_v, sref, tref, chunk_v,
             sem_cp, sem_g, sem_sc):
    cid = lax.axis_index("c")
    sid = lax.axis_index("s")
    wfl = cid * NS + sid

    lo = cid * HALF + sid * CP_ROWS
    cp = pltpu.make_async_copy(
        mem_hbm.at[pl.ds(lo, CP_ROWS)], nm_hbm.at[pl.ds(lo, CP_ROWS)], sem_cp)
    lo_tail = cid * HALF + (NS - 1) * CP_ROWS
    cp_tail = pltpu.make_async_copy(
        mem_hbm.at[pl.ds(lo_tail, CP_TAIL)], nm_hbm.at[pl.ds(lo_tail, CP_TAIL)],
        sem_cp)
    is_tail = sid == NS - 1

    @pl.when(jnp.logical_not(is_tail))
    def _():
        cp.start()

    @pl.when(is_tail)
    def _():
        cp_tail.start()

    @pl.when(cid == 0)
    def _():
        pltpu.sync_copy(gidx_hbm.at[pl.ds(sid * GPW, GPW)], gidx_v)
        pltpu.async_copy(mem_hbm.at[gidx_v], grow_v, sem_g).wait()
        pltpu.sync_copy(grow_v, buf_out.at[pl.ds(sid * GPW, GPW)])

    pltpu.sync_copy(tgt_hbm, tgt_v)
    pltpu.sync_copy(src_hbm, src_v)

    @pl.when(jnp.logical_not(is_tail))
    def _():
        cp.wait()

    @pl.when(is_tail)
    def _():
        cp_tail.wait()

    plsc.subcore_barrier()

    half_lo = cid * HALF
    for k in range(JPS // 16):
        o = sid * JPS + k * 16
        t = tgt_v[pl.ds(o, 16)]
        s = src_v[pl.ds(o, 16)]
        inh = (t >= half_lo) & (t < half_lo + HALF)
        tref[...] = jnp.where(inh, t, lo)
        sref[...] = jnp.where(inh, s, 0)
        pltpu.async_copy(img_hbm.at[sref], chunk_v, sem_sc).wait()
        pltpu.async_copy(chunk_v, nm_hbm.at[tref], sem_sc).wait()

    sref[...] = jnp.zeros((16,), jnp.int32) + wfl
    pltpu.async_copy(fix_hbm.at[sref], chunk_v, sem_sc).wait()
    tref[...] = jnp.zeros((16,), jnp.int32) + lo
    pltpu.async_copy(chunk_v, nm_hbm.at[tref], sem_sc).wait()


@functools.lru_cache(maxsize=1)
def _get_sc_call():
    return functools.partial(
        pl.kernel,
        out_type=(
            jax.ShapeDtypeStruct((K, D), jnp.float32),
            jax.ShapeDtypeStruct((M, D), jnp.float32),
        ),
        mesh=plsc.VectorSubcoreMesh(core_axis_name="c", subcore_axis_name="s"),
        scratch_types=[
            pltpu.VMEM((GPW,), jnp.int32),
            pltpu.VMEM((GPW, D), jnp.float32),
            pltpu.VMEM((BG,), jnp.int32),
            pltpu.VMEM((BG,), jnp.int32),
            pltpu.VMEM((16,), jnp.int32),
            pltpu.VMEM((16,), jnp.int32),
            pltpu.VMEM((16, D), jnp.float32),
            pltpu.SemaphoreType.DMA,
            pltpu.SemaphoreType.DMA,
            pltpu.SemaphoreType.DMA,
        ],
    )(_sc_body)


def kernel(x1, x2, img, mem, W, gather_idx, write_idx, buf_task_labels, task):
    wi = write_idx.astype(jnp.int32)
    src2d, winr2d = _src_call(wi.reshape(BG, 1), wi.reshape(1, BG))
    src = src2d.reshape(BG)
    winr = winr2d.reshape(NW)
    fix = _fix_call(
        winr, mem.reshape(M, 1, D), img.reshape(BG, 1, D)).reshape(NW, D)
    buf, new_mem = _get_sc_call()(
        mem, img, gather_idx.astype(jnp.int32), src, wi, fix)
    loss = _loss_call(x1, x2, buf, W)[0, 0]
    return loss, new_mem
